# baseline (device time: 138480 ns/iter reference)
import jax
import jax.numpy as jnp
from jax import lax
from jax.experimental import pallas as pl
from jax.experimental.pallas import tpu as pltpu

NC = 32
LC_ROWS = 512
LN = 8


def kernel(x):
    m, n = x.shape
    n_out = n // 2
    half = m // 2
    pc = half // NC

    def body(x_ref, o_ref, send_stage,
             gsem, s1send, s1recv, s2send, s2recv, lgsem):
        my_x = lax.axis_index("x")
        my_y = lax.axis_index("y")
        y_nbr = (my_x, 1 - my_y)
        x_nbr = (1 - my_x, my_y)

        barrier_sem = pltpu.get_barrier_semaphore()
        for nbr in (y_nbr, x_nbr):
            pl.semaphore_signal(
                barrier_sem, inc=1, device_id=nbr,
                device_id_type=pl.DeviceIdType.MESH,
            )
        pl.semaphore_wait(barrier_sem, 2)

        local = []
        for c in range(LN):
            g = pltpu.make_async_copy(
                x_ref.at[pl.ds(c * LC_ROWS, LC_ROWS),
                         pl.ds(my_y * n_out, n_out)],
                o_ref.at[pl.ds(my_y * m + c * LC_ROWS, LC_ROWS), :],
                lgsem.at[c],
            )
            g.start()
            local.append(g)

        gathers = []
        for c in range(NC):
            g = pltpu.make_async_copy(
                x_ref.at[pl.ds(my_x * half + c * pc, pc),
                         pl.ds((1 - my_y) * n_out, n_out)],
                send_stage.at[c],
                gsem.at[c],
            )
            g.start()
            gathers.append(g)

        rdma1 = []
        for c in range(NC):
            gathers[c].wait()
            r = pltpu.make_async_remote_copy(
                src_ref=send_stage.at[c],
                dst_ref=o_ref.at[pl.ds(my_y * m + my_x * half + c * pc, pc), :],
                send_sem=s1send.at[c],
                recv_sem=s1recv.at[c],
                device_id=y_nbr,
                device_id_type=pl.DeviceIdType.MESH,
            )
            r.start()
            rdma1.append(r)

        rdma2 = []
        for c in range(NC):
            rdma1[c].wait_recv()
            row0 = (1 - my_y) * m + my_x * half + c * pc
            r = pltpu.make_async_remote_copy(
                src_ref=o_ref.at[pl.ds(row0, pc), :],
                dst_ref=o_ref.at[pl.ds(row0, pc), :],
                send_sem=s2send.at[c],
                recv_sem=s2recv.at[c],
                device_id=x_nbr,
                device_id_type=pl.DeviceIdType.MESH,
            )
            r.start()
            rdma2.append(r)

        for g in local:
            g.wait()
        for c in range(NC):
            rdma1[c].wait_send()
            rdma2[c].wait_recv()
            rdma2[c].wait_send()

    return pl.pallas_call(
        body,
        out_shape=jax.ShapeDtypeStruct((2 * m, n_out), x.dtype),
        in_specs=[pl.BlockSpec(memory_space=pltpu.MemorySpace.HBM)],
        out_specs=pl.BlockSpec(memory_space=pltpu.MemorySpace.VMEM),
        scratch_shapes=[
            pltpu.VMEM((NC, half // NC, n // 2), x.dtype),
            pltpu.SemaphoreType.DMA((NC,)),
            pltpu.SemaphoreType.DMA((NC,)),
            pltpu.SemaphoreType.DMA((NC,)),
            pltpu.SemaphoreType.DMA((NC,)),
            pltpu.SemaphoreType.DMA((NC,)),
            pltpu.SemaphoreType.DMA((LN,)),
        ],
        compiler_params=pltpu.CompilerParams(
            collective_id=0,
            vmem_limit_bytes=56 * 1024 * 1024,
        ),
    )(x)


# device time: 124799 ns/iter; 1.1096x vs baseline; 1.1096x over previous
import jax
import jax.numpy as jnp
from jax import lax
from jax.experimental import pallas as pl
from jax.experimental.pallas import tpu as pltpu

NC = 32
LC_ROWS = 512
LN = 8


def kernel(x):
    m, n = x.shape
    n_out = n // 2
    half = m // 2
    pc = half // NC

    def body(x_ref, o_ref, send_stage, recv_stage, vbuf,
             gsem, s1send, s1recv, s2send, s2recv, stsem, lsem):
        my_x = lax.axis_index("x")
        my_y = lax.axis_index("y")
        y_nbr = (my_x, 1 - my_y)
        x_nbr = (1 - my_x, my_y)

        barrier_sem = pltpu.get_barrier_semaphore()
        for nbr in (y_nbr, x_nbr):
            pl.semaphore_signal(
                barrier_sem, inc=1, device_id=nbr,
                device_id_type=pl.DeviceIdType.MESH,
            )
        pl.semaphore_wait(barrier_sem, 2)

        gathers = []
        for c in range(NC):
            g = pltpu.make_async_copy(
                x_ref.at[pl.ds(my_x * half + c * pc, pc),
                         pl.ds((1 - my_y) * n_out, n_out)],
                send_stage.at[c],
                gsem.at[c],
            )
            g.start()
            gathers.append(g)

        rdma1 = []
        for c in range(NC):
            gathers[c].wait()
            r = pltpu.make_async_remote_copy(
                src_ref=send_stage.at[c],
                dst_ref=recv_stage.at[c],
                send_sem=s1send.at[c],
                recv_sem=s1recv.at[c],
                device_id=y_nbr,
                device_id_type=pl.DeviceIdType.MESH,
            )
            r.start()
            rdma1.append(r)

        stores = []

        def local_chunk(c):
            slot = c % 2
            if c >= 2:
                stores[c - 2].wait()
            g = pltpu.make_async_copy(
                x_ref.at[pl.ds(c * LC_ROWS, LC_ROWS),
                         pl.ds(my_y * n_out, n_out)],
                vbuf.at[slot],
                lsem.at[slot],
            )
            g.start()
            g.wait()
            s = pltpu.make_async_copy(
                vbuf.at[slot],
                o_ref.at[pl.ds(my_y * m + c * LC_ROWS, LC_ROWS), :],
                lsem.at[2 + slot],
            )
            s.start()
            stores.append(s)

        rdma2 = []
        p2stores = []
        for c in range(NC):
            rdma1[c].wait_recv()
            row0 = (1 - my_y) * m + my_x * half + c * pc
            r = pltpu.make_async_remote_copy(
                src_ref=recv_stage.at[c],
                dst_ref=o_ref.at[pl.ds(row0, pc), :],
                send_sem=s2send.at[c],
                recv_sem=s2recv.at[c],
                device_id=x_nbr,
                device_id_type=pl.DeviceIdType.MESH,
            )
            r.start()
            rdma2.append(r)
            st = pltpu.make_async_copy(
                recv_stage.at[c],
                o_ref.at[pl.ds(row0, pc), :],
                stsem.at[c],
            )
            st.start()
            p2stores.append(st)
            if c % (NC // LN) == NC // LN - 1 and c // (NC // LN) < LN:
                local_chunk(c // (NC // LN))

        stores[LN - 2].wait()
        stores[LN - 1].wait()
        for c in range(NC):
            rdma1[c].wait_send()
            rdma2[c].wait_recv()
            rdma2[c].wait_send()
            p2stores[c].wait()

    return pl.pallas_call(
        body,
        out_shape=jax.ShapeDtypeStruct((2 * m, n_out), x.dtype),
        in_specs=[pl.BlockSpec(memory_space=pltpu.MemorySpace.HBM)],
        out_specs=pl.BlockSpec(memory_space=pltpu.MemorySpace.HBM),
        scratch_shapes=[
            pltpu.VMEM((NC, half // NC, n // 2), x.dtype),
            pltpu.VMEM((NC, half // NC, n // 2), x.dtype),
            pltpu.VMEM((2, LC_ROWS, n // 2), x.dtype),
            pltpu.SemaphoreType.DMA((NC,)),
            pltpu.SemaphoreType.DMA((NC,)),
            pltpu.SemaphoreType.DMA((NC,)),
            pltpu.SemaphoreType.DMA((NC,)),
            pltpu.SemaphoreType.DMA((NC,)),
            pltpu.SemaphoreType.DMA((NC,)),
            pltpu.SemaphoreType.DMA((4,)),
        ],
        compiler_params=pltpu.CompilerParams(collective_id=0),
    )(x)
